# baseline (speedup 1.0000x reference)
import functools

import jax
import jax.numpy as jnp
from jax import lax
from jax.experimental import pallas as pl
from jax.experimental.pallas import tpu as pltpu
from jax.experimental.pallas import tpu_sc as plsc

N = 10000
E = 320000
H = 8
D = 128
DH = 128
NPASS = 4
DQ = DH // NPASS

B = 1000
NB = N // B
CH = 160
NCHUNK = E // CH
NTILES = 16
TCH = NCHUNK // NTILES
HC = H // 2
ROWS_T = N // NTILES
DN_PAD = 10240
DNT = DN_PAD // NTILES


def _pre_body(x_ref, w_ref, asrc_ref, adst_ref, h_ref,
              as_ref, ad_ref, ms_ref, md_ref):
    i = pl.program_id(1)
    xb = x_ref[...]
    hb = jnp.dot(xb, w_ref[0], preferred_element_type=jnp.float32)
    h_ref[...] = hb
    asv = jnp.sum(hb * asrc_ref[0], axis=1)
    adv = jnp.sum(hb * adst_ref[0], axis=1)
    as_ref[0, i, :] = asv
    ad_ref[0, i, :] = adv
    msv = jnp.full((1, 1, 128), jnp.max(asv), jnp.float32)
    mdv = jnp.full((1, 1, 128), jnp.max(adv), jnp.float32)

    @pl.when(i == 0)
    def _():
        ms_ref[...] = msv
        md_ref[...] = mdv

    @pl.when(i != 0)
    def _():
        ms_ref[...] = jnp.maximum(ms_ref[...], msv)
        md_ref[...] = jnp.maximum(md_ref[...], mdv)


def _pre_call(x, wr, asrc, adst):
    return pl.pallas_call(
        _pre_body,
        grid=(H, NB),
        in_specs=[
            pl.BlockSpec((B, D), lambda j, i: (i, 0)),
            pl.BlockSpec((1, D, DH), lambda j, i: (j, 0, 0)),
            pl.BlockSpec((1, 1, DH), lambda j, i: (j, 0, 0)),
            pl.BlockSpec((1, 1, DH), lambda j, i: (j, 0, 0)),
        ],
        out_specs=[
            pl.BlockSpec((B, DH), lambda j, i: (j * NB + i, 0)),
            pl.BlockSpec((1, NB, B), lambda j, i: (j, 0, 0)),
            pl.BlockSpec((1, NB, B), lambda j, i: (j, 0, 0)),
            pl.BlockSpec((1, 1, 128), lambda j, i: (j, 0, 0)),
            pl.BlockSpec((1, 1, 128), lambda j, i: (j, 0, 0)),
        ],
        out_shape=[
            jax.ShapeDtypeStruct((H * N, DH), jnp.float32),
            jax.ShapeDtypeStruct((H, NB, B), jnp.float32),
            jax.ShapeDtypeStruct((H, NB, B), jnp.float32),
            jax.ShapeDtypeStruct((H, 1, 128), jnp.float32),
            jax.ShapeDtypeStruct((H, 1, 128), jnp.float32),
        ],
    )(x, wr, asrc, adst)


NBUF = 5
NGRP = TCH // NBUF


def _edge_body(table, as_t, ad_t, ms, md, eb2, zrow, zflat,
               out, dn,
               srcb, dstb, asb, adb, msb, mdb, pbig,
               rows0, rows1, rows2, rows3, rows4,
               ix0, ix1, ix2, ix3, ix4,
               sem0, sem1, sem2, sem3, sem4,
               sb0, sb1, sb2, sb3, sb4, sdn,
               out_sh, dn_sh):
    c = lax.axis_index("c")
    s = lax.axis_index("s")
    rows = [rows0, rows1, rows2, rows3, rows4]
    ixs = [ix0, ix1, ix2, ix3, ix4]
    sems = [sem0, sem1, sem2, sem3, sem4]
    sbs = [sb0, sb1, sb2, sb3, sb4]

    pltpu.sync_copy(eb2.at[s], srcb)

    def unpack(ci, carry):
        for kk in range(CH // 16):
            sl = pl.ds(kk * 16, 16)
            v = srcb[ci, sl]
            dstb[ci, sl] = jax.lax.shift_right_logical(v, 14)
            srcb[ci, sl] = jax.lax.bitwise_and(v, 0x3FFF)
        return carry

    lax.fori_loop(0, TCH, unpack, 0)

    def headloop(k, carry0):
        h = c * HC + k

        def passloop(t, carry):
            pltpu.sync_copy(zrow, out_sh.at[pl.ds(s * ROWS_T, ROWS_T)])

            @pl.when(t == 0)
            def _():
                pltpu.sync_copy(zflat, dn_sh.at[pl.ds(s * DNT, DNT)])
                pltpu.sync_copy(as_t.at[h], asb)
                pltpu.sync_copy(ad_t.at[h], adb)
                pltpu.sync_copy(ms.at[h, 0, pl.ds(0, 16)], msb)
                pltpu.sync_copy(md.at[h, 0, pl.ds(0, 16)], mdb)

            plsc.subcore_barrier()

            base = NPASS * h * N + t

            def stage1(ci, ixb):
                m0 = msb[...] + mdb[...]
                m16 = jnp.where(m0 > 0, m0, 0.2 * m0)
                for kk in range(CH // 16):
                    sl = pl.ds(kk * 16, 16)
                    sv = srcb[ci, sl]
                    dv = dstb[ci, sl]
                    e = (plsc.load_gather(asb, [sv])
                         + plsc.load_gather(adb, [dv]))
                    e = jnp.where(e > 0, e, 0.2 * e)
                    pbig[ci, sl] = jnp.exp(e - m16)
                    ixb[sl] = NPASS * sv + base

            def idx_only(ci, ixb):
                for kk in range(CH // 16):
                    sl = pl.ds(kk * 16, 16)
                    ixb[sl] = NPASS * srcb[ci, sl] + base

            def fire(b, ci, wait_scatter):
                if wait_scatter:
                    pltpu.make_async_copy(
                        rows[b], out_sh.at[dstb.at[ci]], sbs[b]).wait()

                @pl.when(t == 0)
                def _():
                    stage1(ci, ixs[b])

                @pl.when(t != 0)
                def _():
                    idx_only(ci, ixs[b])

                pltpu.make_async_copy(table.at[ixs[b]], rows[b],
                                      sems[b]).start()

            def drain(b, ci):
                pltpu.make_async_copy(table.at[ixs[b]], rows[b],
                                      sems[b]).wait()
                rb = rows[b]

                def emul(g, c3):
                    pvec = pbig[ci, pl.ds(g * 16, 16)]
                    for j in range(16):
                        bv = jnp.full((16,), pvec[j], jnp.float32)
                        ei = g * 16 + j
                        for r in range(DQ // 16):
                            rsl = pl.ds(r * 16, 16)
                            rb[ei, rsl] = rb[ei, rsl] * bv
                    return c3

                lax.fori_loop(0, CH // 16, emul, 0, unroll=2)
                pltpu.async_copy(rb, out_sh.at[dstb.at[ci]], sbs[b],
                                 add=True)

                @pl.when(t == 0)
                def _():
                    @pl.when(ci >= NBUF)
                    def _():
                        pltpu.make_async_copy(
                            pbig.at[ci], dn_sh.at[dstb.at[ci]], sdn).wait()
                    pltpu.async_copy(pbig.at[ci], dn_sh.at[dstb.at[ci]], sdn,
                                     add=True)

            for b in range(NBUF):
                fire(b, b, False)

            def grp(g, c2):
                gbase = g * NBUF
                for b in range(NBUF):
                    drain(b, gbase + b)
                for b in range(NBUF):
                    fire(b, gbase + b + NBUF, True)
                return c2

            lax.fori_loop(0, NGRP - 1, grp, 0)
            for b in range(NBUF):
                drain(b, (NGRP - 1) * NBUF + b)
            for b in range(NBUF):
                ci = (NGRP - 1) * NBUF + b
                pltpu.make_async_copy(rows[b], out_sh.at[dstb.at[ci]],
                                      sbs[b]).wait()

            @pl.when(t == 0)
            def _():
                for b in range(NBUF):
                    ci = TCH - NBUF + b
                    pltpu.make_async_copy(
                        pbig.at[ci], dn_sh.at[dstb.at[ci]], sdn).wait()

            plsc.subcore_barrier()
            pltpu.sync_copy(out_sh.at[pl.ds(s * ROWS_T, ROWS_T)],
                            out.at[t, h, s])

            @pl.when(t == 0)
            def _():
                pltpu.sync_copy(dn_sh.at[pl.ds(s * DNT, DNT)],
                                dn.at[h, pl.ds(s * DNT, DNT)])

            return carry

        lax.fori_loop(0, NPASS, passloop, 0)
        return carry0

    lax.fori_loop(0, HC, headloop, 0)


def _make_edge_fn():
    mesh = plsc.VectorSubcoreMesh(core_axis_name="c", subcore_axis_name="s")
    return pl.kernel(
        _edge_body,
        out_type=[
            jax.ShapeDtypeStruct((NPASS, H, NTILES, ROWS_T, DQ), jnp.float32),
            jax.ShapeDtypeStruct((H, DN_PAD), jnp.float32),
        ],
        mesh=mesh,
        compiler_params=pltpu.CompilerParams(needs_layout_passes=False,
                                             use_tc_tiling_on_sc=False),
        scratch_types=[
            pltpu.VMEM((TCH, CH), jnp.int32),
            pltpu.VMEM((TCH, CH), jnp.int32),
            pltpu.VMEM((N,), jnp.float32),
            pltpu.VMEM((N,), jnp.float32),
            pltpu.VMEM((16,), jnp.float32),
            pltpu.VMEM((16,), jnp.float32),
            pltpu.VMEM((TCH, CH), jnp.float32),
            pltpu.VMEM((CH, DQ), jnp.float32),
            pltpu.VMEM((CH, DQ), jnp.float32),
            pltpu.VMEM((CH, DQ), jnp.float32),
            pltpu.VMEM((CH, DQ), jnp.float32),
            pltpu.VMEM((CH, DQ), jnp.float32),
            pltpu.VMEM((CH,), jnp.int32),
            pltpu.VMEM((CH,), jnp.int32),
            pltpu.VMEM((CH,), jnp.int32),
            pltpu.VMEM((CH,), jnp.int32),
            pltpu.VMEM((CH,), jnp.int32),
            pltpu.SemaphoreType.DMA,
            pltpu.SemaphoreType.DMA,
            pltpu.SemaphoreType.DMA,
            pltpu.SemaphoreType.DMA,
            pltpu.SemaphoreType.DMA,
            pltpu.SemaphoreType.DMA,
            pltpu.SemaphoreType.DMA,
            pltpu.SemaphoreType.DMA,
            pltpu.SemaphoreType.DMA,
            pltpu.SemaphoreType.DMA,
            pltpu.SemaphoreType.DMA,
            pltpu.VMEM_SHARED((N, DQ), jnp.float32),
            pltpu.VMEM_SHARED((DN_PAD,), jnp.float32),
        ],
    )


_EDGE_FN = _make_edge_fn()


def _mean_body(oq_ref, dn_ref, b_ref, o_ref, *, relu):
    i = pl.program_id(0)
    o = oq_ref[...]
    d = dn_ref[:, i, 0, :]
    w = 1.0 / (H * (d[:, :, None] + 1e-16))
    r = jnp.concatenate([jnp.sum(o[t] * w, axis=0) for t in range(NPASS)],
                        axis=-1) + b_ref[0][None, :]
    if relu:
        r = jnp.maximum(r, 0.0)
    o_ref[...] = r


def _mean_call(oq, dnv, b, relu):
    return pl.pallas_call(
        functools.partial(_mean_body, relu=relu),
        grid=(NB,),
        in_specs=[
            pl.BlockSpec((NPASS, H, B, DQ), lambda i: (0, 0, i, 0)),
            pl.BlockSpec((H, NB, 1, B), lambda i: (0, 0, 0, 0)),
            pl.BlockSpec((1, DH), lambda i: (0, 0)),
        ],
        out_specs=pl.BlockSpec((B, DH), lambda i: (i, 0)),
        out_shape=jax.ShapeDtypeStruct((N, DH), jnp.float32),
    )(oq, dnv, b)


def kernel(x, edge_index, edge_weight, W1, att_src1, att_dst1, b1,
           W2, att_src2, att_dst2, b2):
    ei32 = edge_index.astype(jnp.int32)
    eb2 = (ei32[0] | (ei32[1] << 14)).reshape(NTILES, TCH, CH)
    zrow = jnp.zeros((ROWS_T, DQ), jnp.float32)
    zflat = jnp.zeros((DNT,), jnp.float32)

    def layer(xin, W, a_s, a_d, b, relu):
        wr = W.reshape(D, H, DH).transpose(1, 0, 2)
        ht, as3, ad3, ms, md = _pre_call(
            xin, wr, a_s.reshape(H, 1, DH), a_d.reshape(H, 1, DH))
        tq = ht.reshape(NPASS * H * N, DQ)
        oq, dnf = _EDGE_FN(tq, as3.reshape(H, N), ad3.reshape(H, N),
                           ms, md, eb2, zrow, zflat)
        return _mean_call(oq.reshape(NPASS, H, N, DQ),
                          dnf[:, :N].reshape(H, NB, 1, B),
                          b.reshape(1, DH), relu)

    h1 = layer(x, W1, att_src1, att_dst1, b1, True)
    return layer(h1, W2, att_src2, att_dst2, b2, False)


# speedup vs baseline: 2.3486x; 2.3486x over previous
import functools

import jax
import jax.numpy as jnp
from jax import lax
from jax.experimental import pallas as pl
from jax.experimental.pallas import tpu as pltpu
from jax.experimental.pallas import tpu_sc as plsc

N = 10000
E = 320000
H = 8
D = 128
DH = 128
NPASS = 4
DQ = DH // NPASS

B = 1000
NB = N // B
CH = 160
NCHUNK = E // CH
NTILES = 16
TCH = NCHUNK // NTILES
HC = H // 2
ROWS_T = N // NTILES
DN_PAD = 10240
DNT = DN_PAD // NTILES


def _pre_body(x_ref, w_ref, asrc_ref, adst_ref, h_ref,
              as_ref, ad_ref, ms_ref, md_ref):
    i = pl.program_id(1)
    xb = x_ref[...]
    hb = jnp.dot(xb, w_ref[0], preferred_element_type=jnp.float32)
    h_ref[...] = hb
    asv = jnp.sum(hb * asrc_ref[0], axis=1)
    adv = jnp.sum(hb * adst_ref[0], axis=1)
    as_ref[0, i, :] = asv
    ad_ref[0, i, :] = adv
    msv = jnp.full((1, 1, 128), jnp.max(asv), jnp.float32)
    mdv = jnp.full((1, 1, 128), jnp.max(adv), jnp.float32)

    @pl.when(i == 0)
    def _():
        ms_ref[...] = msv
        md_ref[...] = mdv

    @pl.when(i != 0)
    def _():
        ms_ref[...] = jnp.maximum(ms_ref[...], msv)
        md_ref[...] = jnp.maximum(md_ref[...], mdv)


def _pre_call(x, wr, asrc, adst):
    return pl.pallas_call(
        _pre_body,
        grid=(H, NB),
        in_specs=[
            pl.BlockSpec((B, D), lambda j, i: (i, 0)),
            pl.BlockSpec((1, D, DH), lambda j, i: (j, 0, 0)),
            pl.BlockSpec((1, 1, DH), lambda j, i: (j, 0, 0)),
            pl.BlockSpec((1, 1, DH), lambda j, i: (j, 0, 0)),
        ],
        out_specs=[
            pl.BlockSpec((B, DH), lambda j, i: (j * NB + i, 0)),
            pl.BlockSpec((1, NB, B), lambda j, i: (j, 0, 0)),
            pl.BlockSpec((1, NB, B), lambda j, i: (j, 0, 0)),
            pl.BlockSpec((1, 1, 128), lambda j, i: (j, 0, 0)),
            pl.BlockSpec((1, 1, 128), lambda j, i: (j, 0, 0)),
        ],
        out_shape=[
            jax.ShapeDtypeStruct((H * N, DH), jnp.float32),
            jax.ShapeDtypeStruct((H, NB, B), jnp.float32),
            jax.ShapeDtypeStruct((H, NB, B), jnp.float32),
            jax.ShapeDtypeStruct((H, 1, 128), jnp.float32),
            jax.ShapeDtypeStruct((H, 1, 128), jnp.float32),
        ],
    )(x, wr, asrc, adst)


NBUF = 5
NGRP = TCH // NBUF


def _edge_body(table, as_t, ad_t, ms, md, eb2, zrow, zflat,
               out, dn,
               srcb, dstb, asb, adb, msb, mdb, pbig,
               rows0, rows1, rows2, rows3, rows4,
               ix0, ix1, ix2, ix3, ix4,
               sem0, sem1, sem2, sem3, sem4,
               sb0, sb1, sb2, sb3, sb4, sdn,
               out_sh, dn_sh):
    c = lax.axis_index("c")
    s = lax.axis_index("s")
    rows = [rows0, rows1, rows2, rows3, rows4]
    ixs = [ix0, ix1, ix2, ix3, ix4]
    sems = [sem0, sem1, sem2, sem3, sem4]
    sbs = [sb0, sb1, sb2, sb3, sb4]

    pltpu.sync_copy(eb2.at[s], srcb)

    def unpack(ci, carry):
        for kk in range(CH // 16):
            sl = pl.ds(kk * 16, 16)
            v = srcb[ci, sl]
            dstb[ci, sl] = jax.lax.shift_right_logical(v, 14)
            srcb[ci, sl] = jax.lax.bitwise_and(v, 0x3FFF)
        return carry

    lax.fori_loop(0, TCH, unpack, 0)

    def headloop(k, carry0):
        h = c * HC + k

        def passloop(t, carry):
            pltpu.sync_copy(zrow, out_sh.at[pl.ds(s * ROWS_T, ROWS_T)])

            @pl.when(t == 0)
            def _():
                pltpu.sync_copy(zflat, dn_sh.at[pl.ds(s * DNT, DNT)])
                pltpu.sync_copy(as_t.at[h], asb)
                pltpu.sync_copy(ad_t.at[h], adb)
                pltpu.sync_copy(ms.at[h, 0, pl.ds(0, 16)], msb)
                pltpu.sync_copy(md.at[h, 0, pl.ds(0, 16)], mdb)

            plsc.subcore_barrier()

            base = NPASS * h * N + t

            def stage1(ci, ixb):
                m0 = msb[...] + mdb[...]
                m16 = jnp.where(m0 > 0, m0, 0.2 * m0)
                for kk in range(CH // 16):
                    sl = pl.ds(kk * 16, 16)
                    sv = srcb[ci, sl]
                    dv = dstb[ci, sl]
                    e = (plsc.load_gather(asb, [sv])
                         + plsc.load_gather(adb, [dv]))
                    e = jnp.where(e > 0, e, 0.2 * e)
                    pbig[ci, sl] = jnp.exp(e - m16)
                    ixb[sl] = NPASS * sv + base

            def idx_only(ci, ixb):
                for kk in range(CH // 16):
                    sl = pl.ds(kk * 16, 16)
                    ixb[sl] = NPASS * srcb[ci, sl] + base

            def fire(b, ci, wait_scatter):
                if wait_scatter:
                    pltpu.make_async_copy(
                        rows[b], out_sh.at[dstb.at[ci]], sbs[b]).wait()

                @pl.when(t == 0)
                def _():
                    stage1(ci, ixs[b])

                @pl.when(t != 0)
                def _():
                    idx_only(ci, ixs[b])

                pltpu.make_async_copy(table.at[ixs[b]], rows[b],
                                      sems[b]).start()

            def drain(b, ci):
                pltpu.make_async_copy(table.at[ixs[b]], rows[b],
                                      sems[b]).wait()
                rb = rows[b]

                def emul(g, c3):
                    pvec = pbig[ci, pl.ds(g * 16, 16)]
                    for j in range(16):
                        bv = jnp.full((16,), pvec[j], jnp.float32)
                        ei = g * 16 + j
                        for r in range(DQ // 16):
                            rsl = pl.ds(r * 16, 16)
                            rb[ei, rsl] = rb[ei, rsl] * bv
                    return c3

                lax.fori_loop(0, CH // 16, emul, 0)
                pltpu.async_copy(rb, out_sh.at[dstb.at[ci]], sbs[b],
                                 add=True)

                @pl.when(t == 0)
                def _():
                    @pl.when(ci >= NBUF)
                    def _():
                        pltpu.make_async_copy(
                            pbig.at[ci], dn_sh.at[dstb.at[ci]], sdn).wait()
                    pltpu.async_copy(pbig.at[ci], dn_sh.at[dstb.at[ci]], sdn,
                                     add=True)

            for b in range(NBUF):
                fire(b, b, False)

            def grp(g, c2):
                gbase = g * NBUF
                for b in range(NBUF):
                    drain(b, gbase + b)
                for b in range(NBUF):
                    fire(b, gbase + b + NBUF, True)
                return c2

            lax.fori_loop(0, NGRP - 1, grp, 0)
            for b in range(NBUF):
                drain(b, (NGRP - 1) * NBUF + b)
            for b in range(NBUF):
                ci = (NGRP - 1) * NBUF + b
                pltpu.make_async_copy(rows[b], out_sh.at[dstb.at[ci]],
                                      sbs[b]).wait()

            @pl.when(t == 0)
            def _():
                for b in range(NBUF):
                    ci = TCH - NBUF + b
                    pltpu.make_async_copy(
                        pbig.at[ci], dn_sh.at[dstb.at[ci]], sdn).wait()

            plsc.subcore_barrier()
            pltpu.sync_copy(out_sh.at[pl.ds(s * ROWS_T, ROWS_T)],
                            out.at[t, h, s])

            @pl.when(t == 0)
            def _():
                pltpu.sync_copy(dn_sh.at[pl.ds(s * DNT, DNT)],
                                dn.at[h, pl.ds(s * DNT, DNT)])

            return carry

        lax.fori_loop(0, NPASS, passloop, 0)
        return carry0

    lax.fori_loop(0, HC, headloop, 0)


def _make_edge_fn():
    mesh = plsc.VectorSubcoreMesh(core_axis_name="c", subcore_axis_name="s")
    return pl.kernel(
        _edge_body,
        out_type=[
            jax.ShapeDtypeStruct((NPASS, H, NTILES, ROWS_T, DQ), jnp.float32),
            jax.ShapeDtypeStruct((H, DN_PAD), jnp.float32),
        ],
        mesh=mesh,
        compiler_params=pltpu.CompilerParams(needs_layout_passes=False,
                                             use_tc_tiling_on_sc=False),
        scratch_types=[
            pltpu.VMEM((TCH, CH), jnp.int32),
            pltpu.VMEM((TCH, CH), jnp.int32),
            pltpu.VMEM((N,), jnp.float32),
            pltpu.VMEM((N,), jnp.float32),
            pltpu.VMEM((16,), jnp.float32),
            pltpu.VMEM((16,), jnp.float32),
            pltpu.VMEM((TCH, CH), jnp.float32),
            pltpu.VMEM((CH, DQ), jnp.float32),
            pltpu.VMEM((CH, DQ), jnp.float32),
            pltpu.VMEM((CH, DQ), jnp.float32),
            pltpu.VMEM((CH, DQ), jnp.float32),
            pltpu.VMEM((CH, DQ), jnp.float32),
            pltpu.VMEM((CH,), jnp.int32),
            pltpu.VMEM((CH,), jnp.int32),
            pltpu.VMEM((CH,), jnp.int32),
            pltpu.VMEM((CH,), jnp.int32),
            pltpu.VMEM((CH,), jnp.int32),
            pltpu.SemaphoreType.DMA,
            pltpu.SemaphoreType.DMA,
            pltpu.SemaphoreType.DMA,
            pltpu.SemaphoreType.DMA,
            pltpu.SemaphoreType.DMA,
            pltpu.SemaphoreType.DMA,
            pltpu.SemaphoreType.DMA,
            pltpu.SemaphoreType.DMA,
            pltpu.SemaphoreType.DMA,
            pltpu.SemaphoreType.DMA,
            pltpu.SemaphoreType.DMA,
            pltpu.VMEM_SHARED((N, DQ), jnp.float32),
            pltpu.VMEM_SHARED((DN_PAD,), jnp.float32),
        ],
    )


_EDGE_FN = _make_edge_fn()


def _mean_body(oq_ref, dn_ref, b_ref, o_ref, *, relu):
    i = pl.program_id(0)
    o = oq_ref[...]
    d = dn_ref[:, i, 0, :]
    w = 1.0 / (H * (d[:, :, None] + 1e-16))
    r = jnp.concatenate([jnp.sum(o[t] * w, axis=0) for t in range(NPASS)],
                        axis=-1) + b_ref[0][None, :]
    if relu:
        r = jnp.maximum(r, 0.0)
    o_ref[...] = r


def _mean_call(oq, dnv, b, relu):
    return pl.pallas_call(
        functools.partial(_mean_body, relu=relu),
        grid=(NB,),
        in_specs=[
            pl.BlockSpec((NPASS, H, B, DQ), lambda i: (0, 0, i, 0)),
            pl.BlockSpec((H, NB, 1, B), lambda i: (0, 0, 0, 0)),
            pl.BlockSpec((1, DH), lambda i: (0, 0)),
        ],
        out_specs=pl.BlockSpec((B, DH), lambda i: (i, 0)),
        out_shape=jax.ShapeDtypeStruct((N, DH), jnp.float32),
    )(oq, dnv, b)


def kernel(x, edge_index, edge_weight, W1, att_src1, att_dst1, b1,
           W2, att_src2, att_dst2, b2):
    ei32 = edge_index.astype(jnp.int32)
    eb2 = (ei32[0] | (ei32[1] << 14)).reshape(NTILES, TCH, CH)
    zrow = jnp.zeros((ROWS_T, DQ), jnp.float32)
    zflat = jnp.zeros((DNT,), jnp.float32)

    def layer(xin, W, a_s, a_d, b, relu):
        wr = W.reshape(D, H, DH).transpose(1, 0, 2)
        ht, as3, ad3, ms, md = _pre_call(
            xin, wr, a_s.reshape(H, 1, DH), a_d.reshape(H, 1, DH))
        tq = ht.reshape(NPASS * H * N, DQ)
        oq, dnf = _EDGE_FN(tq, as3.reshape(H, N), ad3.reshape(H, N),
                           ms, md, eb2, zrow, zflat)
        return _mean_call(oq.reshape(NPASS, H, N, DQ),
                          dnf[:, :N].reshape(H, NB, 1, B),
                          b.reshape(1, DH), relu)

    h1 = layer(x, W1, att_src1, att_dst1, b1, True)
    return layer(h1, W2, att_src2, att_dst2, b2, False)


# speedup vs baseline: 2.3548x; 1.0027x over previous
import functools

import jax
import jax.numpy as jnp
from jax import lax
from jax.experimental import pallas as pl
from jax.experimental.pallas import tpu as pltpu
from jax.experimental.pallas import tpu_sc as plsc

N = 10000
E = 320000
H = 8
D = 128
DH = 128
NPASS = 4
DQ = DH // NPASS

B = 1000
NB = N // B
CH = 160
NCHUNK = E // CH
NTILES = 16
TCH = NCHUNK // NTILES
HC = H // 2
ROWS_T = N // NTILES
DN_PAD = 10240
DNT = DN_PAD // NTILES


def _pre_body(x_ref, w_ref, asrc_ref, adst_ref, h_ref,
              as_ref, ad_ref, ms_ref, md_ref):
    i = pl.program_id(1)
    xb = x_ref[...]
    hb = jnp.dot(xb, w_ref[0], preferred_element_type=jnp.float32)
    h_ref[...] = hb
    asv = jnp.sum(hb * asrc_ref[0], axis=1)
    adv = jnp.sum(hb * adst_ref[0], axis=1)
    as_ref[0, i, :] = asv
    ad_ref[0, i, :] = adv
    msv = jnp.full((1, 1, 128), jnp.max(asv), jnp.float32)
    mdv = jnp.full((1, 1, 128), jnp.max(adv), jnp.float32)

    @pl.when(i == 0)
    def _():
        ms_ref[...] = msv
        md_ref[...] = mdv

    @pl.when(i != 0)
    def _():
        ms_ref[...] = jnp.maximum(ms_ref[...], msv)
        md_ref[...] = jnp.maximum(md_ref[...], mdv)


def _pre_call(x, wr, asrc, adst):
    return pl.pallas_call(
        _pre_body,
        grid=(H, NB),
        in_specs=[
            pl.BlockSpec((B, D), lambda j, i: (i, 0)),
            pl.BlockSpec((1, D, DH), lambda j, i: (j, 0, 0)),
            pl.BlockSpec((1, 1, DH), lambda j, i: (j, 0, 0)),
            pl.BlockSpec((1, 1, DH), lambda j, i: (j, 0, 0)),
        ],
        out_specs=[
            pl.BlockSpec((B, DH), lambda j, i: (j * NB + i, 0)),
            pl.BlockSpec((1, NB, B), lambda j, i: (j, 0, 0)),
            pl.BlockSpec((1, NB, B), lambda j, i: (j, 0, 0)),
            pl.BlockSpec((1, 1, 128), lambda j, i: (j, 0, 0)),
            pl.BlockSpec((1, 1, 128), lambda j, i: (j, 0, 0)),
        ],
        out_shape=[
            jax.ShapeDtypeStruct((H * N, DH), jnp.float32),
            jax.ShapeDtypeStruct((H, NB, B), jnp.float32),
            jax.ShapeDtypeStruct((H, NB, B), jnp.float32),
            jax.ShapeDtypeStruct((H, 1, 128), jnp.float32),
            jax.ShapeDtypeStruct((H, 1, 128), jnp.float32),
        ],
    )(x, wr, asrc, adst)


NBUF = 5
NGRP = TCH // NBUF


def _edge_body(table, as_t, ad_t, ms, md, eb2, zrow, zflat,
               out, dn,
               srcb, dstb, asb, adb, msb, mdb, pbig,
               rows0, rows1, rows2, rows3, rows4,
               ix0, ix1, ix2, ix3, ix4,
               sem0, sem1, sem2, sem3, sem4,
               sb0, sb1, sb2, sb3, sb4, sdn, sz,
               out_sh, dn_sh):
    c = lax.axis_index("c")
    s = lax.axis_index("s")
    rows = [rows0, rows1, rows2, rows3, rows4]
    ixs = [ix0, ix1, ix2, ix3, ix4]
    sems = [sem0, sem1, sem2, sem3, sem4]
    sbs = [sb0, sb1, sb2, sb3, sb4]

    pltpu.sync_copy(eb2.at[s], srcb)
    pltpu.make_async_copy(zrow, out_sh.at[pl.ds(s * ROWS_T, ROWS_T)],
                          sz).start()

    def unpack(ci, carry):
        for kk in range(CH // 16):
            sl = pl.ds(kk * 16, 16)
            v = srcb[ci, sl]
            dstb[ci, sl] = jax.lax.shift_right_logical(v, 14)
            srcb[ci, sl] = jax.lax.bitwise_and(v, 0x3FFF)
        return carry

    lax.fori_loop(0, TCH, unpack, 0)

    def headloop(k, carry0):
        h = c * HC + k

        def passloop(t, carry):
            pltpu.make_async_copy(zrow, out_sh.at[pl.ds(s * ROWS_T, ROWS_T)],
                                  sz).wait()

            @pl.when(t == 0)
            def _():
                pltpu.sync_copy(zflat, dn_sh.at[pl.ds(s * DNT, DNT)])
                pltpu.sync_copy(as_t.at[h], asb)
                pltpu.sync_copy(ad_t.at[h], adb)
                pltpu.sync_copy(ms.at[h, 0, pl.ds(0, 16)], msb)
                pltpu.sync_copy(md.at[h, 0, pl.ds(0, 16)], mdb)

            plsc.subcore_barrier()

            base = NPASS * h * N + t

            def stage1(ci, ixb):
                m0 = msb[...] + mdb[...]
                m16 = jnp.where(m0 > 0, m0, 0.2 * m0)
                for kk in range(CH // 16):
                    sl = pl.ds(kk * 16, 16)
                    sv = srcb[ci, sl]
                    dv = dstb[ci, sl]
                    e = (plsc.load_gather(asb, [sv])
                         + plsc.load_gather(adb, [dv]))
                    e = jnp.where(e > 0, e, 0.2 * e)
                    pbig[ci, sl] = jnp.exp(e - m16)
                    ixb[sl] = NPASS * sv + base

            def idx_only(ci, ixb):
                for kk in range(CH // 16):
                    sl = pl.ds(kk * 16, 16)
                    ixb[sl] = NPASS * srcb[ci, sl] + base

            def fire(b, ci, wait_scatter):
                if wait_scatter:
                    pltpu.make_async_copy(
                        rows[b], out_sh.at[dstb.at[ci]], sbs[b]).wait()

                @pl.when(t == 0)
                def _():
                    stage1(ci, ixs[b])

                @pl.when(t != 0)
                def _():
                    idx_only(ci, ixs[b])

                pltpu.make_async_copy(table.at[ixs[b]], rows[b],
                                      sems[b]).start()

            def drain(b, ci):
                pltpu.make_async_copy(table.at[ixs[b]], rows[b],
                                      sems[b]).wait()
                rb = rows[b]

                def emul(g, c3):
                    pvec = pbig[ci, pl.ds(g * 16, 16)]
                    for j in range(16):
                        bv = jnp.full((16,), pvec[j], jnp.float32)
                        ei = g * 16 + j
                        for r in range(DQ // 16):
                            rsl = pl.ds(r * 16, 16)
                            rb[ei, rsl] = rb[ei, rsl] * bv
                    return c3

                lax.fori_loop(0, CH // 16, emul, 0)
                pltpu.async_copy(rb, out_sh.at[dstb.at[ci]], sbs[b],
                                 add=True)

                @pl.when(t == 0)
                def _():
                    @pl.when(ci >= NBUF)
                    def _():
                        pltpu.make_async_copy(
                            pbig.at[ci], dn_sh.at[dstb.at[ci]], sdn).wait()
                    pltpu.async_copy(pbig.at[ci], dn_sh.at[dstb.at[ci]], sdn,
                                     add=True)

            for b in range(NBUF):
                fire(b, b, False)

            def grp(g, c2):
                gbase = g * NBUF
                for b in range(NBUF):
                    drain(b, gbase + b)
                for b in range(NBUF):
                    fire(b, gbase + b + NBUF, True)
                return c2

            lax.fori_loop(0, NGRP - 1, grp, 0)
            for b in range(NBUF):
                drain(b, (NGRP - 1) * NBUF + b)
            for b in range(NBUF):
                ci = (NGRP - 1) * NBUF + b
                pltpu.make_async_copy(rows[b], out_sh.at[dstb.at[ci]],
                                      sbs[b]).wait()

            @pl.when(t == 0)
            def _():
                for b in range(NBUF):
                    ci = TCH - NBUF + b
                    pltpu.make_async_copy(
                        pbig.at[ci], dn_sh.at[dstb.at[ci]], sdn).wait()

            plsc.subcore_barrier()
            pltpu.sync_copy(out_sh.at[pl.ds(s * ROWS_T, ROWS_T)],
                            out.at[t, h, s])

            @pl.when((k < HC - 1) | (t < NPASS - 1))
            def _():
                pltpu.make_async_copy(
                    zrow, out_sh.at[pl.ds(s * ROWS_T, ROWS_T)], sz).start()

            @pl.when(t == 0)
            def _():
                pltpu.sync_copy(dn_sh.at[pl.ds(s * DNT, DNT)],
                                dn.at[h, pl.ds(s * DNT, DNT)])

            return carry

        lax.fori_loop(0, NPASS, passloop, 0)
        return carry0

    lax.fori_loop(0, HC, headloop, 0)


def _make_edge_fn():
    mesh = plsc.VectorSubcoreMesh(core_axis_name="c", subcore_axis_name="s")
    return pl.kernel(
        _edge_body,
        out_type=[
            jax.ShapeDtypeStruct((NPASS, H, NTILES, ROWS_T, DQ), jnp.float32),
            jax.ShapeDtypeStruct((H, DN_PAD), jnp.float32),
        ],
        mesh=mesh,
        compiler_params=pltpu.CompilerParams(needs_layout_passes=False,
                                             use_tc_tiling_on_sc=False),
        scratch_types=[
            pltpu.VMEM((TCH, CH), jnp.int32),
            pltpu.VMEM((TCH, CH), jnp.int32),
            pltpu.VMEM((N,), jnp.float32),
            pltpu.VMEM((N,), jnp.float32),
            pltpu.VMEM((16,), jnp.float32),
            pltpu.VMEM((16,), jnp.float32),
            pltpu.VMEM((TCH, CH), jnp.float32),
            pltpu.VMEM((CH, DQ), jnp.float32),
            pltpu.VMEM((CH, DQ), jnp.float32),
            pltpu.VMEM((CH, DQ), jnp.float32),
            pltpu.VMEM((CH, DQ), jnp.float32),
            pltpu.VMEM((CH, DQ), jnp.float32),
            pltpu.VMEM((CH,), jnp.int32),
            pltpu.VMEM((CH,), jnp.int32),
            pltpu.VMEM((CH,), jnp.int32),
            pltpu.VMEM((CH,), jnp.int32),
            pltpu.VMEM((CH,), jnp.int32),
            pltpu.SemaphoreType.DMA,
            pltpu.SemaphoreType.DMA,
            pltpu.SemaphoreType.DMA,
            pltpu.SemaphoreType.DMA,
            pltpu.SemaphoreType.DMA,
            pltpu.SemaphoreType.DMA,
            pltpu.SemaphoreType.DMA,
            pltpu.SemaphoreType.DMA,
            pltpu.SemaphoreType.DMA,
            pltpu.SemaphoreType.DMA,
            pltpu.SemaphoreType.DMA,
            pltpu.SemaphoreType.DMA,
            pltpu.VMEM_SHARED((N, DQ), jnp.float32),
            pltpu.VMEM_SHARED((DN_PAD,), jnp.float32),
        ],
    )


_EDGE_FN = _make_edge_fn()


def _mean_body(oq_ref, dn_ref, b_ref, o_ref, *, relu):
    i = pl.program_id(0)
    o = oq_ref[...]
    d = dn_ref[:, i, 0, :]
    w = 1.0 / (H * (d[:, :, None] + 1e-16))
    r = jnp.concatenate([jnp.sum(o[t] * w, axis=0) for t in range(NPASS)],
                        axis=-1) + b_ref[0][None, :]
    if relu:
        r = jnp.maximum(r, 0.0)
    o_ref[...] = r


def _mean_call(oq, dnv, b, relu):
    return pl.pallas_call(
        functools.partial(_mean_body, relu=relu),
        grid=(NB,),
        in_specs=[
            pl.BlockSpec((NPASS, H, B, DQ), lambda i: (0, 0, i, 0)),
            pl.BlockSpec((H, NB, 1, B), lambda i: (0, 0, 0, 0)),
            pl.BlockSpec((1, DH), lambda i: (0, 0)),
        ],
        out_specs=pl.BlockSpec((B, DH), lambda i: (i, 0)),
        out_shape=jax.ShapeDtypeStruct((N, DH), jnp.float32),
    )(oq, dnv, b)


def kernel(x, edge_index, edge_weight, W1, att_src1, att_dst1, b1,
           W2, att_src2, att_dst2, b2):
    ei32 = edge_index.astype(jnp.int32)
    eb2 = (ei32[0] | (ei32[1] << 14)).reshape(NTILES, TCH, CH)
    zrow = jnp.zeros((ROWS_T, DQ), jnp.float32)
    zflat = jnp.zeros((DNT,), jnp.float32)

    def layer(xin, W, a_s, a_d, b, relu):
        wr = W.reshape(D, H, DH).transpose(1, 0, 2)
        ht, as3, ad3, ms, md = _pre_call(
            xin, wr, a_s.reshape(H, 1, DH), a_d.reshape(H, 1, DH))
        tq = ht.reshape(NPASS * H * N, DQ)
        oq, dnf = _EDGE_FN(tq, as3.reshape(H, N), ad3.reshape(H, N),
                           ms, md, eb2, zrow, zflat)
        return _mean_call(oq.reshape(NPASS, H, N, DQ),
                          dnf[:, :N].reshape(H, NB, 1, B),
                          b.reshape(1, DH), relu)

    h1 = layer(x, W1, att_src1, att_dst1, b1, True)
    return layer(h1, W2, att_src2, att_dst2, b2, False)
